# Initial kernel scaffold; baseline (speedup 1.0000x reference)
import jax
import jax.numpy as jnp
from jax.experimental import pallas as pl


def kernel(atom_fea, nbr_fea, nbr_idx, pos, W1, b1, W2, b2, w_tp):
    raise NotImplementedError("write your pallas kernel here")


# speedup vs baseline: 6.6427x; 6.6427x over previous
import functools
import math

import jax
import jax.numpy as jnp
from jax import lax
from jax.experimental import pallas as pl
from jax.experimental.pallas import tpu as pltpu
from jax.experimental.pallas import tpu_sc as plsc

N = 10000
M = 32
C = 128
F = 16

NC = 2
NS = 16
NWORK = NC * NS

STEP_NODES = 4
EDGES_STEP = STEP_NODES * M
PW = 320
NPAD = NWORK * PW
NSTEP = PW // STEP_NODES
LANES = 16
KV = C // LANES


def _dense_body(atom_ref, nbrf_ref, w_tp_ref, w1b_ref, b1_ref, w2b_ref, b2c_ref,
                p_ref, r_ref, *, scale):
    p_ref[...] = jnp.dot(atom_ref[...], w_tp_ref[...],
                         preferred_element_type=jnp.float32) * scale
    h = jnp.dot(nbrf_ref[...], w1b_ref[...],
                preferred_element_type=jnp.float32) + b1_ref[...]
    sp = jnp.maximum(h, 0.0) + jnp.log1p(jnp.exp(-jnp.abs(h)))
    r_ref[...] = jnp.dot(sp, w2b_ref[...],
                         preferred_element_type=jnp.float32) + b2c_ref[...]


def _dense_stage(atom_fea, nbr2d, w_tp, W1blk, b1t, w2blk, b2c, scale):
    BA = 2000
    grid = (N // BA,)
    MF = M * F
    return pl.pallas_call(
        functools.partial(_dense_body, scale=scale),
        grid=grid,
        in_specs=[
            pl.BlockSpec((BA, C), lambda i: (i, 0)),
            pl.BlockSpec((BA, MF), lambda i: (i, 0)),
            pl.BlockSpec((C, C), lambda i: (0, 0)),
            pl.BlockSpec((MF, MF), lambda i: (0, 0)),
            pl.BlockSpec((1, MF), lambda i: (0, 0)),
            pl.BlockSpec((MF, M), lambda i: (0, 0)),
            pl.BlockSpec((1, 1), lambda i: (0, 0)),
        ],
        out_specs=[
            pl.BlockSpec((BA, C), lambda i: (i, 0)),
            pl.BlockSpec((BA, M), lambda i: (i, 0)),
        ],
        out_shape=[
            jax.ShapeDtypeStruct((N, C), jnp.float32),
            jax.ShapeDtypeStruct((N, M), jnp.float32),
        ],
    )(atom_fea, nbr2d, w_tp, W1blk, b1t, w2blk, b2c)


def _sc_body(p_hbm, idx_hbm, w_hbm, out_hbm, idx_v, w_v, rows_v, out_v, sem):
    wid = lax.axis_index("s") * NC + lax.axis_index("c")
    base_e = wid * (PW * M)
    pltpu.sync_copy(idx_hbm.at[pl.ds(base_e, PW * M)], idx_v)
    pltpu.sync_copy(w_hbm.at[pl.ds(base_e, PW * M)], w_v)

    def step(g, carry):
        pltpu.async_copy(
            p_hbm.at[idx_v.at[pl.ds(g * EDGES_STEP, EDGES_STEP)]],
            rows_v, sem).wait()
        dnums = lax.GatherDimensionNumbers(
            offset_dims=(), collapsed_slice_dims=(0,), start_index_map=(0,))
        for n_loc in range(STEP_NODES):
            e0 = n_loc * M
            acc = [jnp.zeros((LANES,), jnp.float32) for _ in range(KV)]
            wregs = [w_v[pl.ds(g * EDGES_STEP + e0 + LANES * h, LANES)]
                     for h in range(M // LANES)]
            for j in range(M):
                bidx = jnp.full((LANES, 1), j % LANES, dtype=jnp.int32)
                wv = lax.gather(wregs[j // LANES], bidx, dnums, (1,),
                                mode=lax.GatherScatterMode.PROMISE_IN_BOUNDS)
                for k in range(KV):
                    acc[k] = acc[k] + rows_v[e0 + j, pl.ds(LANES * k, LANES)] * wv
            node = g * STEP_NODES + n_loc
            for k in range(KV):
                out_v[node, pl.ds(LANES * k, LANES)] = acc[k]
        return carry

    lax.fori_loop(0, NSTEP, step, 0)
    pltpu.sync_copy(out_v, out_hbm.at[pl.ds(wid * PW, PW)])


def _sc_stage(P, idx_pad, w_pad):
    mesh = plsc.VectorSubcoreMesh(core_axis_name="c", subcore_axis_name="s")
    k = functools.partial(
        pl.kernel, mesh=mesh,
        out_type=jax.ShapeDtypeStruct((NPAD, C), jnp.float32),
        scratch_types=[
            pltpu.VMEM((PW * M,), jnp.int32),
            pltpu.VMEM((PW * M,), jnp.float32),
            pltpu.VMEM((EDGES_STEP, C), jnp.float32),
            pltpu.VMEM((PW, C), jnp.float32),
            pltpu.SemaphoreType.DMA,
        ],
    )(_sc_body)
    return k(P, idx_pad, w_pad)


def kernel(atom_fea, nbr_fea, nbr_idx, pos, W1, b1, W2, b2, w_tp):
    del pos
    scale = 1.0 / (math.sqrt(4.0 * math.pi) * math.sqrt(C) * M)
    nbr2d = nbr_fea.reshape(N, M * F)
    eye_m = jnp.eye(M, dtype=jnp.float32)
    W1blk = jnp.kron(eye_m, W1)
    w2blk = jnp.kron(eye_m, W2[:, 0:1])
    b1t = jnp.tile(b1, (M,)).reshape(1, M * F)
    b2c = b2[0:1].reshape(1, 1)
    P, R0 = _dense_stage(atom_fea, nbr2d, w_tp, W1blk, b1t, w2blk, b2c, scale)
    pad = NPAD * M - N * M
    idx_pad = jnp.pad(nbr_idx.reshape(-1), (0, pad))
    w_pad = jnp.pad(R0.reshape(-1), (0, pad))
    out = _sc_stage(P, idx_pad, w_pad)
    return out[:N]
